# baseline (speedup 1.0000x reference)
import functools

import jax
import jax.numpy as jnp
import numpy as np
from jax import lax
from jax.experimental import pallas as pl
from jax.experimental.pallas import tpu as pltpu
from jax.experimental.pallas import tpu_sc as plsc

B, L, D = 1024, 20, 10000
VOCAB, NUM_CLASSES, NGRAM_N = 1000, 100, 3

NC, NS = 2, 16
NW = NC * NS
BPW = B // NW
RB = 8

NCHUNK = 5
DP = 10240
DC = DP // NCHUNK
WB = DC + 32

_NT = L - (NGRAM_N - 1)


def _build_tables(id_weight):
    wb = id_weight.astype(jnp.bfloat16)
    pad = (NCHUNK - 1) * DC + 1 + WB - 2 - D
    ext = jnp.concatenate([wb[:, D - 2 :], wb, jnp.zeros((VOCAB, pad), wb.dtype)], 1)
    tabs = []
    for shift in (0, 1):
        wins = [
            lax.dynamic_slice_in_dim(ext, c * DC + shift, WB, 1)
            for c in range(NCHUNK)
        ]
        tabs.append(jnp.stack(wins, axis=1).reshape(VOCAB * NCHUNK, WB))
    return tabs


def _sc_encode(table_a, table_b, x):
    mesh = plsc.VectorSubcoreMesh(
        core_axis_name="c", subcore_axis_name="s", num_cores=NC, num_subcores=NS
    )

    @functools.partial(
        pl.kernel,
        out_type=jax.ShapeDtypeStruct((B, DP), jnp.bfloat16),
        mesh=mesh,
        compiler_params=pltpu.CompilerParams(use_tc_tiling_on_sc=False),
        scratch_types=[
            pltpu.VMEM((BPW, L), jnp.int32),
            pltpu.VMEM((2, L), jnp.int32),
            pltpu.VMEM((2, L, WB), jnp.bfloat16),
            pltpu.VMEM((2, L, WB), jnp.bfloat16),
            pltpu.VMEM((RB, DC), jnp.bfloat16),
            pltpu.SemaphoreType.DMA,
            pltpu.SemaphoreType.DMA,
        ],
    )
    def enc_kernel(
        ta_hbm, tb_hbm, x_hbm, enc_hbm, xw, idxv, bufa, bufb, acc, sem0, sem1
    ):
        wid = lax.axis_index("s") * NC + lax.axis_index("c")
        base_b = wid * BPW
        pltpu.sync_copy(x_hbm.at[pl.ds(base_b, BPW)], xw)
        sems = (sem0, sem1)

        def fire(pb, i, c):
            idxv[pb, pl.ds(0, 16)] = xw[i, pl.ds(0, 16)] * NCHUNK + c
            idxv[pb, pl.ds(4, 16)] = xw[i, pl.ds(4, 16)] * NCHUNK + c
            pltpu.async_copy(ta_hbm.at[idxv.at[pb]], bufa.at[pb], sems[pb])
            pltpu.async_copy(tb_hbm.at[idxv.at[pb]], bufb.at[pb], sems[pb])

        def drain(pb):
            pltpu.make_async_copy(ta_hbm.at[idxv.at[pb]], bufa.at[pb], sems[pb]).wait()
            pltpu.make_async_copy(tb_hbm.at[idxv.at[pb]], bufb.at[pb], sems[pb]).wait()

        def compute(pb, r):
            for g in range(DC // 32):
                base = g * 32

                def tbody(t, a):
                    v = bufa[pb, t, pl.ds(base, 32)]
                    v = v * bufb[pb, t + 1, pl.ds(base, 32)]
                    v = v * bufa[pb, t + 2, pl.ds(base + 2, 32)]
                    return a + v

                acc[r, pl.ds(base, 32)] = lax.fori_loop(
                    0, _NT, tbody, jnp.zeros((32,), jnp.bfloat16), unroll=2
                )

        def body_grp(i8, carry):
            def body_c(c, carry2):
                fire(0, i8 * RB, c)

                def body_r2(r2, carry3):
                    r0 = r2 * 2
                    fire(1, i8 * RB + r0 + 1, c)
                    drain(0)
                    compute(0, r0)

                    @pl.when(r2 < RB // 2 - 1)
                    def _():
                        fire(0, i8 * RB + r0 + 2, c)

                    drain(1)
                    compute(1, r0 + 1)
                    return carry3

                lax.fori_loop(0, RB // 2, body_r2, 0)
                row0 = pl.multiple_of(base_b + i8 * RB, RB)
                col0 = pl.multiple_of(c * DC, 256)
                pltpu.sync_copy(
                    acc, enc_hbm.at[pl.ds(row0, RB), pl.ds(col0, DC)]
                )
                return carry2

            lax.fori_loop(0, NCHUNK, body_c, 0)
            return carry

        lax.fori_loop(0, BPW // RB, body_grp, 0)

    return enc_kernel(table_a, table_b, x)


def _tc_search(enc, am_pad):
    BB = 128

    def body(enc_ref, am_ref, out_ref):
        am = am_ref[...]
        an = jnp.sqrt(jnp.sum(am * am, axis=1, keepdims=True)) + 1e-12
        am_n = am / an
        e = enc_ref[...].astype(jnp.float32)
        en = jnp.sqrt(jnp.sum(e * e, axis=1, keepdims=True)) + 1e-12
        s = lax.dot_general(
            e, am_n, (((1,), (1,)), ((), ())), preferred_element_type=jnp.float32
        )
        out_ref[...] = s / en

    return pl.pallas_call(
        body,
        grid=(B // BB,),
        in_specs=[
            pl.BlockSpec((BB, DP), lambda i: (i, 0)),
            pl.BlockSpec((NUM_CLASSES, DP), lambda i: (0, 0)),
        ],
        out_specs=pl.BlockSpec((BB, NUM_CLASSES), lambda i: (i, 0)),
        out_shape=jax.ShapeDtypeStruct((B, NUM_CLASSES), jnp.float32),
    )(enc, am_pad)


@jax.jit
def kernel(x, id_weight, am_weight):
    table_a, table_b = _build_tables(id_weight)
    enc = _sc_encode(table_a, table_b, x.astype(jnp.int32))
    am_pad = jnp.pad(am_weight, ((0, 0), (0, DP - D)))
    return _tc_search(enc, am_pad)


# speedup vs baseline: 1.4845x; 1.4845x over previous
import functools

import jax
import jax.numpy as jnp
import numpy as np
from jax import lax
from jax.experimental import pallas as pl
from jax.experimental.pallas import tpu as pltpu
from jax.experimental.pallas import tpu_sc as plsc

B, L, D = 1024, 20, 10000
VOCAB, NUM_CLASSES, NGRAM_N = 1000, 100, 3

NC, NS = 2, 16
NW = NC * NS
BPW = B // NW
RB = 8

NCHUNK = 5
DP = 10240
DC = DP // NCHUNK
WB = DC + 32

_NT = L - (NGRAM_N - 1)


def _build_tables(id_weight):
    wb = id_weight.astype(jnp.bfloat16)
    pad = (NCHUNK - 1) * DC + 1 + WB - 2 - D
    ext = jnp.concatenate([wb[:, D - 2 :], wb, jnp.zeros((VOCAB, pad), wb.dtype)], 1)
    tabs = []
    for shift in (0, 1):
        wins = [
            lax.dynamic_slice_in_dim(ext, c * DC + shift, WB, 1)
            for c in range(NCHUNK)
        ]
        tabs.append(jnp.stack(wins, axis=1).reshape(VOCAB * NCHUNK, WB))
    return tabs


def _sc_encode(table_a, table_b, x):
    mesh = plsc.VectorSubcoreMesh(
        core_axis_name="c", subcore_axis_name="s", num_cores=NC, num_subcores=NS
    )

    @functools.partial(
        pl.kernel,
        out_type=jax.ShapeDtypeStruct((B, DP), jnp.bfloat16),
        mesh=mesh,
        compiler_params=pltpu.CompilerParams(use_tc_tiling_on_sc=False),
        scratch_types=[
            pltpu.VMEM((BPW, L), jnp.int32),
            pltpu.VMEM((2, L), jnp.int32),
            pltpu.VMEM((2, L, WB), jnp.bfloat16),
            pltpu.VMEM((2, L, WB), jnp.bfloat16),
            pltpu.VMEM((RB, DC), jnp.bfloat16),
            pltpu.SemaphoreType.DMA,
            pltpu.SemaphoreType.DMA,
        ],
    )
    def enc_kernel(
        ta_hbm, tb_hbm, x_hbm, enc_hbm, xw, idxv, bufa, bufb, acc, sem0, sem1
    ):
        wid = lax.axis_index("s") * NC + lax.axis_index("c")
        base_b = wid * BPW
        pltpu.sync_copy(x_hbm.at[pl.ds(base_b, BPW)], xw)
        sems = (sem0, sem1)

        def fire(pb, i, c):
            idxv[pb, pl.ds(0, 16)] = xw[i, pl.ds(0, 16)] * NCHUNK + c
            idxv[pb, pl.ds(4, 16)] = xw[i, pl.ds(4, 16)] * NCHUNK + c
            pltpu.async_copy(ta_hbm.at[idxv.at[pb]], bufa.at[pb], sems[pb])
            pltpu.async_copy(tb_hbm.at[idxv.at[pb]], bufb.at[pb], sems[pb])

        def drain(pb):
            pltpu.make_async_copy(ta_hbm.at[idxv.at[pb]], bufa.at[pb], sems[pb]).wait()
            pltpu.make_async_copy(tb_hbm.at[idxv.at[pb]], bufb.at[pb], sems[pb]).wait()

        def compute(pb, r):
            for g in range(DC // 32):
                base = g * 32

                def tbody(t, a):
                    v = bufa[pb, t, pl.ds(base, 32)]
                    v = v * bufb[pb, t + 1, pl.ds(base, 32)]
                    v = v * bufa[pb, t + 2, pl.ds(base + 2, 32)]
                    return a + v

                acc[r, pl.ds(base, 32)] = lax.fori_loop(
                    0, _NT, tbody, jnp.zeros((32,), jnp.bfloat16)
                )

        def body_grp(i8, carry):
            def body_c(c, carry2):
                fire(0, i8 * RB, c)

                def body_r2(r2, carry3):
                    r0 = r2 * 2
                    fire(1, i8 * RB + r0 + 1, c)
                    drain(0)
                    compute(0, r0)

                    @pl.when(r2 < RB // 2 - 1)
                    def _():
                        fire(0, i8 * RB + r0 + 2, c)

                    drain(1)
                    compute(1, r0 + 1)
                    return carry3

                lax.fori_loop(0, RB // 2, body_r2, 0)
                row0 = pl.multiple_of(base_b + i8 * RB, RB)
                col0 = pl.multiple_of(c * DC, 256)
                pltpu.sync_copy(
                    acc, enc_hbm.at[pl.ds(row0, RB), pl.ds(col0, DC)]
                )
                return carry2

            lax.fori_loop(0, NCHUNK, body_c, 0)
            return carry

        lax.fori_loop(0, BPW // RB, body_grp, 0)

    return enc_kernel(table_a, table_b, x)


def _tc_search(enc, am_pad):
    BB = 128

    def body(enc_ref, am_ref, out_ref):
        am = am_ref[...]
        an = jnp.sqrt(jnp.sum(am * am, axis=1, keepdims=True)) + 1e-12
        am_n = am / an
        e = enc_ref[...].astype(jnp.float32)
        en = jnp.sqrt(jnp.sum(e * e, axis=1, keepdims=True)) + 1e-12
        s = lax.dot_general(
            e, am_n, (((1,), (1,)), ((), ())), preferred_element_type=jnp.float32
        )
        out_ref[...] = s / en

    return pl.pallas_call(
        body,
        grid=(B // BB,),
        in_specs=[
            pl.BlockSpec((BB, DP), lambda i: (i, 0)),
            pl.BlockSpec((NUM_CLASSES, DP), lambda i: (0, 0)),
        ],
        out_specs=pl.BlockSpec((BB, NUM_CLASSES), lambda i: (i, 0)),
        out_shape=jax.ShapeDtypeStruct((B, NUM_CLASSES), jnp.float32),
    )(enc, am_pad)


@jax.jit
def kernel(x, id_weight, am_weight):
    table_a, table_b = _build_tables(id_weight)
    enc = _sc_encode(table_a, table_b, x.astype(jnp.int32))
    am_pad = jnp.pad(am_weight, ((0, 0), (0, DP - D)))
    return _tc_search(enc, am_pad)


# speedup vs baseline: 1.9527x; 1.3154x over previous
import functools

import jax
import jax.numpy as jnp
import numpy as np
from jax import lax
from jax.experimental import pallas as pl
from jax.experimental.pallas import tpu as pltpu
from jax.experimental.pallas import tpu_sc as plsc

B, L, D = 1024, 20, 10000
VOCAB, NUM_CLASSES, NGRAM_N = 1000, 100, 3

NC, NS = 2, 16
NW = NC * NS
BPW = B // NW
RB = 8

NCHUNK = 5
DP = 10240
DC = DP // NCHUNK
WB = DC + 32

_NT = L - (NGRAM_N - 1)


def _build_tables(id_weight):
    wb = id_weight.astype(jnp.bfloat16)
    pad = (NCHUNK - 1) * DC + 1 + WB - 2 - D
    ext = jnp.concatenate([wb[:, D - 2 :], wb, jnp.zeros((VOCAB, pad), wb.dtype)], 1)
    tabs = []
    for shift in (0, 1):
        wins = [
            lax.dynamic_slice_in_dim(ext, c * DC + shift, WB, 1)
            for c in range(NCHUNK)
        ]
        tabs.append(jnp.stack(wins, axis=1).reshape(VOCAB * NCHUNK, WB))
    return tabs


def _sc_encode(table_a, table_b, x):
    mesh = plsc.VectorSubcoreMesh(
        core_axis_name="c", subcore_axis_name="s", num_cores=NC, num_subcores=NS
    )

    @functools.partial(
        pl.kernel,
        out_type=jax.ShapeDtypeStruct((B, DP), jnp.bfloat16),
        mesh=mesh,
        compiler_params=pltpu.CompilerParams(use_tc_tiling_on_sc=False),
        scratch_types=[
            pltpu.VMEM((BPW, L), jnp.int32),
            pltpu.VMEM((2, L), jnp.int32),
            pltpu.VMEM((2, L, WB), jnp.bfloat16),
            pltpu.VMEM((2, L, WB), jnp.bfloat16),
            pltpu.VMEM((RB, DC), jnp.bfloat16),
            pltpu.SemaphoreType.DMA,
            pltpu.SemaphoreType.DMA,
        ],
    )
    def enc_kernel(
        ta_hbm, tb_hbm, x_hbm, enc_hbm, xw, idxv, bufa, bufb, acc, sem0, sem1
    ):
        wid = lax.axis_index("s") * NC + lax.axis_index("c")
        base_b = wid * BPW
        pltpu.sync_copy(x_hbm.at[pl.ds(base_b, BPW)], xw)
        sems = (sem0, sem1)

        def fire(pb, i, c):
            idxv[pb, pl.ds(0, 16)] = xw[i, pl.ds(0, 16)] * NCHUNK + c
            idxv[pb, pl.ds(4, 16)] = xw[i, pl.ds(4, 16)] * NCHUNK + c
            pltpu.async_copy(ta_hbm.at[idxv.at[pb]], bufa.at[pb], sems[pb])
            pltpu.async_copy(tb_hbm.at[idxv.at[pb]], bufb.at[pb], sems[pb])

        def drain(pb):
            pltpu.make_async_copy(ta_hbm.at[idxv.at[pb]], bufa.at[pb], sems[pb]).wait()
            pltpu.make_async_copy(tb_hbm.at[idxv.at[pb]], bufb.at[pb], sems[pb]).wait()

        def compute(pb, r):
            for g in range(DC // 64):
                base = g * 64

                def tbody(t, ab):
                    a0, a1 = ab
                    v0 = bufa[pb, t, pl.ds(base, 32)]
                    v0 = v0 * bufb[pb, t + 1, pl.ds(base, 32)]
                    v0 = v0 * bufa[pb, t + 2, pl.ds(base + 2, 32)]
                    v1 = bufa[pb, t, pl.ds(base + 32, 32)]
                    v1 = v1 * bufb[pb, t + 1, pl.ds(base + 32, 32)]
                    v1 = v1 * bufa[pb, t + 2, pl.ds(base + 34, 32)]
                    return (a0 + v0, a1 + v1)

                z = jnp.zeros((32,), jnp.bfloat16)
                a0, a1 = lax.fori_loop(0, _NT, tbody, (z, z))
                acc[r, pl.ds(base, 32)] = a0
                acc[r, pl.ds(base + 32, 32)] = a1

        def body_grp(i8, carry):
            def body_c(c, carry2):
                fire(0, i8 * RB, c)

                def body_r2(r2, carry3):
                    r0 = r2 * 2
                    fire(1, i8 * RB + r0 + 1, c)
                    drain(0)
                    compute(0, r0)

                    @pl.when(r2 < RB // 2 - 1)
                    def _():
                        fire(0, i8 * RB + r0 + 2, c)

                    drain(1)
                    compute(1, r0 + 1)
                    return carry3

                lax.fori_loop(0, RB // 2, body_r2, 0)
                row0 = pl.multiple_of(base_b + i8 * RB, RB)
                col0 = pl.multiple_of(c * DC, 256)
                pltpu.sync_copy(
                    acc, enc_hbm.at[pl.ds(row0, RB), pl.ds(col0, DC)]
                )
                return carry2

            lax.fori_loop(0, NCHUNK, body_c, 0)
            return carry

        lax.fori_loop(0, BPW // RB, body_grp, 0)

    return enc_kernel(table_a, table_b, x)


def _tc_search(enc, am_pad):
    BB = 128

    def body(enc_ref, am_ref, out_ref):
        am = am_ref[...]
        an = jnp.sqrt(jnp.sum(am * am, axis=1, keepdims=True)) + 1e-12
        am_n = am / an
        e = enc_ref[...].astype(jnp.float32)
        en = jnp.sqrt(jnp.sum(e * e, axis=1, keepdims=True)) + 1e-12
        s = lax.dot_general(
            e, am_n, (((1,), (1,)), ((), ())), preferred_element_type=jnp.float32
        )
        out_ref[...] = s / en

    return pl.pallas_call(
        body,
        grid=(B // BB,),
        in_specs=[
            pl.BlockSpec((BB, DP), lambda i: (i, 0)),
            pl.BlockSpec((NUM_CLASSES, DP), lambda i: (0, 0)),
        ],
        out_specs=pl.BlockSpec((BB, NUM_CLASSES), lambda i: (i, 0)),
        out_shape=jax.ShapeDtypeStruct((B, NUM_CLASSES), jnp.float32),
    )(enc, am_pad)


@jax.jit
def kernel(x, id_weight, am_weight):
    table_a, table_b = _build_tables(id_weight)
    enc = _sc_encode(table_a, table_b, x.astype(jnp.int32))
    am_pad = jnp.pad(am_weight, ((0, 0), (0, DP - D)))
    return _tc_search(enc, am_pad)


# speedup vs baseline: 2.0114x; 1.0300x over previous
import functools

import jax
import jax.numpy as jnp
import numpy as np
from jax import lax
from jax.experimental import pallas as pl
from jax.experimental.pallas import tpu as pltpu
from jax.experimental.pallas import tpu_sc as plsc

B, L, D = 1024, 20, 10000
VOCAB, NUM_CLASSES, NGRAM_N = 1000, 100, 3

NC, NS = 2, 16
NW = NC * NS
BPW = B // NW
RB = 8

NCHUNK = 5
DP = 10240
DC = DP // NCHUNK
WB = DC + 32

_NT = L - (NGRAM_N - 1)


def _build_tables(id_weight):
    wb = id_weight.astype(jnp.bfloat16)
    pad = (NCHUNK - 1) * DC + 1 + WB - 2 - D
    ext = jnp.concatenate([wb[:, D - 2 :], wb, jnp.zeros((VOCAB, pad), wb.dtype)], 1)
    tabs = []
    for shift in (0, 1):
        wins = [
            lax.dynamic_slice_in_dim(ext, c * DC + shift, WB, 1)
            for c in range(NCHUNK)
        ]
        tabs.append(jnp.stack(wins, axis=1).reshape(VOCAB * NCHUNK, WB))
    return tabs


def _sc_encode(table_a, table_b, x):
    mesh = plsc.VectorSubcoreMesh(
        core_axis_name="c", subcore_axis_name="s", num_cores=NC, num_subcores=NS
    )

    @functools.partial(
        pl.kernel,
        out_type=jax.ShapeDtypeStruct((B, DP), jnp.bfloat16),
        mesh=mesh,
        compiler_params=pltpu.CompilerParams(use_tc_tiling_on_sc=False),
        scratch_types=[
            pltpu.VMEM((BPW, L), jnp.int32),
            pltpu.VMEM((2, L), jnp.int32),
            pltpu.VMEM((2, L, WB), jnp.bfloat16),
            pltpu.VMEM((2, L, WB), jnp.bfloat16),
            pltpu.VMEM((RB, DC), jnp.bfloat16),
            pltpu.SemaphoreType.DMA,
            pltpu.SemaphoreType.DMA,
        ],
    )
    def enc_kernel(
        ta_hbm, tb_hbm, x_hbm, enc_hbm, xw, idxv, bufa, bufb, acc, sem0, sem1
    ):
        wid = lax.axis_index("s") * NC + lax.axis_index("c")
        base_b = wid * BPW
        pltpu.sync_copy(x_hbm.at[pl.ds(base_b, BPW)], xw)
        sems = (sem0, sem1)

        def fire(pb, i, c):
            idxv[pb, pl.ds(0, 16)] = xw[i, pl.ds(0, 16)] * NCHUNK + c
            idxv[pb, pl.ds(4, 16)] = xw[i, pl.ds(4, 16)] * NCHUNK + c
            pltpu.async_copy(ta_hbm.at[idxv.at[pb]], bufa.at[pb], sems[pb])
            pltpu.async_copy(tb_hbm.at[idxv.at[pb]], bufb.at[pb], sems[pb])

        def drain(pb):
            pltpu.make_async_copy(ta_hbm.at[idxv.at[pb]], bufa.at[pb], sems[pb]).wait()
            pltpu.make_async_copy(tb_hbm.at[idxv.at[pb]], bufb.at[pb], sems[pb]).wait()

        def compute(pb, r):
            NG = 4
            for g in range(DC // (32 * NG)):
                base = g * 32 * NG

                def tbody(t, accs):
                    out = []
                    for k in range(NG):
                        o = base + k * 32
                        v = bufa[pb, t, pl.ds(o, 32)]
                        v = v * bufb[pb, t + 1, pl.ds(o, 32)]
                        v = v * bufa[pb, t + 2, pl.ds(o + 2, 32)]
                        out.append(accs[k] + v)
                    return tuple(out)

                z = jnp.zeros((32,), jnp.bfloat16)
                accs = lax.fori_loop(0, _NT, tbody, (z,) * NG)
                for k in range(NG):
                    acc[r, pl.ds(base + k * 32, 32)] = accs[k]

        def body_grp(i8, carry):
            def body_c(c, carry2):
                fire(0, i8 * RB, c)

                def body_r2(r2, carry3):
                    r0 = r2 * 2
                    fire(1, i8 * RB + r0 + 1, c)
                    drain(0)
                    compute(0, r0)

                    @pl.when(r2 < RB // 2 - 1)
                    def _():
                        fire(0, i8 * RB + r0 + 2, c)

                    drain(1)
                    compute(1, r0 + 1)
                    return carry3

                lax.fori_loop(0, RB // 2, body_r2, 0)
                row0 = pl.multiple_of(base_b + i8 * RB, RB)
                col0 = pl.multiple_of(c * DC, 256)
                pltpu.sync_copy(
                    acc, enc_hbm.at[pl.ds(row0, RB), pl.ds(col0, DC)]
                )
                return carry2

            lax.fori_loop(0, NCHUNK, body_c, 0)
            return carry

        lax.fori_loop(0, BPW // RB, body_grp, 0)

    return enc_kernel(table_a, table_b, x)


def _tc_search(enc, am_pad):
    BB = 128

    def body(enc_ref, am_ref, out_ref):
        am = am_ref[...]
        an = jnp.sqrt(jnp.sum(am * am, axis=1, keepdims=True)) + 1e-12
        am_n = am / an
        e = enc_ref[...].astype(jnp.float32)
        en = jnp.sqrt(jnp.sum(e * e, axis=1, keepdims=True)) + 1e-12
        s = lax.dot_general(
            e, am_n, (((1,), (1,)), ((), ())), preferred_element_type=jnp.float32
        )
        out_ref[...] = s / en

    return pl.pallas_call(
        body,
        grid=(B // BB,),
        in_specs=[
            pl.BlockSpec((BB, DP), lambda i: (i, 0)),
            pl.BlockSpec((NUM_CLASSES, DP), lambda i: (0, 0)),
        ],
        out_specs=pl.BlockSpec((BB, NUM_CLASSES), lambda i: (i, 0)),
        out_shape=jax.ShapeDtypeStruct((B, NUM_CLASSES), jnp.float32),
    )(enc, am_pad)


@jax.jit
def kernel(x, id_weight, am_weight):
    table_a, table_b = _build_tables(id_weight)
    enc = _sc_encode(table_a, table_b, x.astype(jnp.int32))
    am_pad = jnp.pad(am_weight, ((0, 0), (0, DP - D)))
    return _tc_search(enc, am_pad)


# speedup vs baseline: 2.0302x; 1.0094x over previous
import functools

import jax
import jax.numpy as jnp
import numpy as np
from jax import lax
from jax.experimental import pallas as pl
from jax.experimental.pallas import tpu as pltpu
from jax.experimental.pallas import tpu_sc as plsc

B, L, D = 1024, 20, 10000
VOCAB, NUM_CLASSES, NGRAM_N = 1000, 100, 3

NC, NS = 2, 16
NW = NC * NS
BPW = B // NW
RB = 8

NCHUNK = 5
DP = 10240
DC = DP // NCHUNK
WB = DC + 32

_NT = L - (NGRAM_N - 1)


def _build_tables(id_weight):
    wb = id_weight.astype(jnp.bfloat16)
    pad = (NCHUNK - 1) * DC + 1 + WB - 2 - D
    ext = jnp.concatenate([wb[:, D - 2 :], wb, jnp.zeros((VOCAB, pad), wb.dtype)], 1)
    tabs = []
    for shift in (0, 1):
        wins = [
            lax.dynamic_slice_in_dim(ext, c * DC + shift, WB, 1)
            for c in range(NCHUNK)
        ]
        tabs.append(jnp.stack(wins, axis=1).reshape(VOCAB * NCHUNK, WB))
    return tabs


def _sc_encode(table_a, table_b, x):
    mesh = plsc.VectorSubcoreMesh(
        core_axis_name="c", subcore_axis_name="s", num_cores=NC, num_subcores=NS
    )

    @functools.partial(
        pl.kernel,
        out_type=jax.ShapeDtypeStruct((B, DP), jnp.bfloat16),
        mesh=mesh,
        compiler_params=pltpu.CompilerParams(use_tc_tiling_on_sc=False),
        scratch_types=[
            pltpu.VMEM((BPW, L), jnp.int32),
            pltpu.VMEM((2, L), jnp.int32),
            pltpu.VMEM((2, L, WB), jnp.bfloat16),
            pltpu.VMEM((2, L, WB), jnp.bfloat16),
            pltpu.VMEM((RB, DC), jnp.bfloat16),
            pltpu.SemaphoreType.DMA,
            pltpu.SemaphoreType.DMA,
        ],
    )
    def enc_kernel(
        ta_hbm, tb_hbm, x_hbm, enc_hbm, xw, idxv, bufa, bufb, acc, sem0, sem1
    ):
        wid = lax.axis_index("s") * NC + lax.axis_index("c")
        base_b = wid * BPW
        pltpu.sync_copy(x_hbm.at[pl.ds(base_b, BPW)], xw)
        sems = (sem0, sem1)

        def fire(pb, i, c):
            idxv[pb, pl.ds(0, 16)] = xw[i, pl.ds(0, 16)] * NCHUNK + c
            idxv[pb, pl.ds(4, 16)] = xw[i, pl.ds(4, 16)] * NCHUNK + c
            pltpu.async_copy(ta_hbm.at[idxv.at[pb]], bufa.at[pb], sems[pb])
            pltpu.async_copy(tb_hbm.at[idxv.at[pb]], bufb.at[pb], sems[pb])

        def drain(pb):
            pltpu.make_async_copy(ta_hbm.at[idxv.at[pb]], bufa.at[pb], sems[pb]).wait()
            pltpu.make_async_copy(tb_hbm.at[idxv.at[pb]], bufb.at[pb], sems[pb]).wait()

        def compute(pb, r):
            NG = 8
            for g in range(DC // (32 * NG)):
                base = g * 32 * NG

                def tbody(t, accs):
                    out = []
                    for k in range(NG):
                        o = base + k * 32
                        v = bufa[pb, t, pl.ds(o, 32)]
                        v = v * bufb[pb, t + 1, pl.ds(o, 32)]
                        v = v * bufa[pb, t + 2, pl.ds(o + 2, 32)]
                        out.append(accs[k] + v)
                    return tuple(out)

                z = jnp.zeros((32,), jnp.bfloat16)
                accs = lax.fori_loop(0, _NT, tbody, (z,) * NG)
                for k in range(NG):
                    acc[r, pl.ds(base + k * 32, 32)] = accs[k]

        def body_grp(i8, carry):
            def body_c(c, carry2):
                fire(0, i8 * RB, c)

                def body_r2(r2, carry3):
                    r0 = r2 * 2
                    fire(1, i8 * RB + r0 + 1, c)
                    drain(0)
                    compute(0, r0)

                    @pl.when(r2 < RB // 2 - 1)
                    def _():
                        fire(0, i8 * RB + r0 + 2, c)

                    drain(1)
                    compute(1, r0 + 1)
                    return carry3

                lax.fori_loop(0, RB // 2, body_r2, 0)
                row0 = pl.multiple_of(base_b + i8 * RB, RB)
                col0 = pl.multiple_of(c * DC, 256)
                pltpu.sync_copy(
                    acc, enc_hbm.at[pl.ds(row0, RB), pl.ds(col0, DC)]
                )
                return carry2

            lax.fori_loop(0, NCHUNK, body_c, 0)
            return carry

        lax.fori_loop(0, BPW // RB, body_grp, 0)

    return enc_kernel(table_a, table_b, x)


def _tc_search(enc, am_pad):
    BB = 128

    def body(enc_ref, am_ref, out_ref):
        am = am_ref[...]
        an = jnp.sqrt(jnp.sum(am * am, axis=1, keepdims=True)) + 1e-12
        am_n = am / an
        e = enc_ref[...].astype(jnp.float32)
        en = jnp.sqrt(jnp.sum(e * e, axis=1, keepdims=True)) + 1e-12
        s = lax.dot_general(
            e, am_n, (((1,), (1,)), ((), ())), preferred_element_type=jnp.float32
        )
        out_ref[...] = s / en

    return pl.pallas_call(
        body,
        grid=(B // BB,),
        in_specs=[
            pl.BlockSpec((BB, DP), lambda i: (i, 0)),
            pl.BlockSpec((NUM_CLASSES, DP), lambda i: (0, 0)),
        ],
        out_specs=pl.BlockSpec((BB, NUM_CLASSES), lambda i: (i, 0)),
        out_shape=jax.ShapeDtypeStruct((B, NUM_CLASSES), jnp.float32),
    )(enc, am_pad)


@jax.jit
def kernel(x, id_weight, am_weight):
    table_a, table_b = _build_tables(id_weight)
    enc = _sc_encode(table_a, table_b, x.astype(jnp.int32))
    am_pad = jnp.pad(am_weight, ((0, 0), (0, DP - D)))
    return _tc_search(enc, am_pad)


# speedup vs baseline: 2.1960x; 1.0817x over previous
import functools

import jax
import jax.numpy as jnp
import numpy as np
from jax import lax
from jax.experimental import pallas as pl
from jax.experimental.pallas import tpu as pltpu
from jax.experimental.pallas import tpu_sc as plsc

B, L, D = 1024, 20, 10000
VOCAB, NUM_CLASSES, NGRAM_N = 1000, 100, 3

NC, NS = 2, 16
NW = NC * NS
BPW = B // NW
RB = 8

NCHUNK = 5
DP = 10240
DC = DP // NCHUNK
WB = DC + 32

_NT = L - (NGRAM_N - 1)


def _build_tables(id_weight):
    wb = id_weight.astype(jnp.bfloat16)
    pad = (NCHUNK - 1) * DC + 1 + WB - 2 - D
    ext = jnp.concatenate([wb[:, D - 2 :], wb, jnp.zeros((VOCAB, pad), wb.dtype)], 1)
    tabs = []
    for shift in (0, 1):
        wins = [
            lax.dynamic_slice_in_dim(ext, c * DC + shift, WB, 1)
            for c in range(NCHUNK)
        ]
        tabs.append(jnp.stack(wins, axis=1).reshape(VOCAB * NCHUNK, WB))
    return tabs


def _sc_encode(table_a, table_b, x):
    mesh = plsc.VectorSubcoreMesh(
        core_axis_name="c", subcore_axis_name="s", num_cores=NC, num_subcores=NS
    )

    @functools.partial(
        pl.kernel,
        out_type=jax.ShapeDtypeStruct((B, DP), jnp.bfloat16),
        mesh=mesh,
        compiler_params=pltpu.CompilerParams(use_tc_tiling_on_sc=False),
        scratch_types=[
            pltpu.VMEM((BPW, L), jnp.int32),
            pltpu.VMEM((2, L), jnp.int32),
            pltpu.VMEM((2, L, WB), jnp.bfloat16),
            pltpu.VMEM((2, L, WB), jnp.bfloat16),
            pltpu.VMEM((RB, DC), jnp.bfloat16),
            pltpu.SemaphoreType.DMA,
            pltpu.SemaphoreType.DMA,
        ],
    )
    def enc_kernel(
        ta_hbm, tb_hbm, x_hbm, enc_hbm, xw, idxv, bufa, bufb, acc, sem0, sem1
    ):
        wid = lax.axis_index("s") * NC + lax.axis_index("c")
        base_b = wid * BPW
        pltpu.sync_copy(x_hbm.at[pl.ds(base_b, BPW)], xw)
        sems = (sem0, sem1)

        def fire(pb, i, c):
            idxv[pb, pl.ds(0, 16)] = xw[i, pl.ds(0, 16)] * NCHUNK + c
            idxv[pb, pl.ds(4, 16)] = xw[i, pl.ds(4, 16)] * NCHUNK + c
            pltpu.async_copy(ta_hbm.at[idxv.at[pb]], bufa.at[pb], sems[pb])
            pltpu.async_copy(tb_hbm.at[idxv.at[pb]], bufb.at[pb], sems[pb])

        def drain(pb):
            pltpu.make_async_copy(ta_hbm.at[idxv.at[pb]], bufa.at[pb], sems[pb]).wait()
            pltpu.make_async_copy(tb_hbm.at[idxv.at[pb]], bufb.at[pb], sems[pb]).wait()

        def compute(pb, r):
            NG = 8
            for g in range(DC // (32 * NG)):
                base = g * 32 * NG

                def tbody(t, accs):
                    out = []
                    for k in range(NG):
                        o = base + k * 32
                        v = bufa[pb, t, pl.ds(o, 32)]
                        v = v * bufb[pb, t + 1, pl.ds(o, 32)]
                        v = v * bufa[pb, t + 2, pl.ds(o + 2, 32)]
                        out.append(accs[k] + v)
                    return tuple(out)

                z = jnp.zeros((32,), jnp.bfloat16)
                accs = lax.fori_loop(0, _NT, tbody, (z,) * NG)
                for k in range(NG):
                    acc[r, pl.ds(base + k * 32, 32)] = accs[k]

        NR2 = RB // 2
        NU = (BPW // RB) * NCHUNK * NR2

        def unit(u):
            i8 = u // (NCHUNK * NR2)
            rem = u % (NCHUNK * NR2)
            return i8, rem // NR2, (rem % NR2) * 2

        fire(0, 0, 0)

        def body_u(u, carry):
            i8, c, r0 = unit(u)
            i = i8 * RB + r0
            fire(1, i + 1, c)
            drain(0)
            compute(0, r0)

            @pl.when(u < NU - 1)
            def _():
                ni8, nc, nr0 = unit(u + 1)
                fire(0, ni8 * RB + nr0, nc)

            drain(1)
            compute(1, r0 + 1)

            @pl.when(r0 == RB - 2)
            def _():
                row0 = pl.multiple_of(base_b + i8 * RB, RB)
                col0 = pl.multiple_of(c * DC, 256)
                pltpu.sync_copy(
                    acc, enc_hbm.at[pl.ds(row0, RB), pl.ds(col0, DC)]
                )

            return carry

        lax.fori_loop(0, NU, body_u, 0)

    return enc_kernel(table_a, table_b, x)


def _tc_search(enc, am_pad):
    BB = 128

    def body(enc_ref, am_ref, out_ref):
        am = am_ref[...]
        an = jnp.sqrt(jnp.sum(am * am, axis=1, keepdims=True)) + 1e-12
        am_n = am / an
        e = enc_ref[...].astype(jnp.float32)
        en = jnp.sqrt(jnp.sum(e * e, axis=1, keepdims=True)) + 1e-12
        s = lax.dot_general(
            e, am_n, (((1,), (1,)), ((), ())), preferred_element_type=jnp.float32
        )
        out_ref[...] = s / en

    return pl.pallas_call(
        body,
        grid=(B // BB,),
        in_specs=[
            pl.BlockSpec((BB, DP), lambda i: (i, 0)),
            pl.BlockSpec((NUM_CLASSES, DP), lambda i: (0, 0)),
        ],
        out_specs=pl.BlockSpec((BB, NUM_CLASSES), lambda i: (i, 0)),
        out_shape=jax.ShapeDtypeStruct((B, NUM_CLASSES), jnp.float32),
    )(enc, am_pad)


@jax.jit
def kernel(x, id_weight, am_weight):
    table_a, table_b = _build_tables(id_weight)
    enc = _sc_encode(table_a, table_b, x.astype(jnp.int32))
    am_pad = jnp.pad(am_weight, ((0, 0), (0, DP - D)))
    return _tc_search(enc, am_pad)


# speedup vs baseline: 2.1964x; 1.0002x over previous
import functools

import jax
import jax.numpy as jnp
import numpy as np
from jax import lax
from jax.experimental import pallas as pl
from jax.experimental.pallas import tpu as pltpu
from jax.experimental.pallas import tpu_sc as plsc

B, L, D = 1024, 20, 10000
VOCAB, NUM_CLASSES, NGRAM_N = 1000, 100, 3

NC, NS = 2, 16
NW = NC * NS
BPW = B // NW
RB = 8

NCHUNK = 5
DP = 10240
DC = DP // NCHUNK
WB = DC + 32

_NT = L - (NGRAM_N - 1)


def _build_tables(id_weight):
    wb = id_weight.astype(jnp.bfloat16)
    pad = (NCHUNK - 1) * DC + 1 + WB - 2 - D
    ext = jnp.concatenate([wb[:, D - 2 :], wb, jnp.zeros((VOCAB, pad), wb.dtype)], 1)
    tabs = []
    for shift in (0, 1):
        wins = [
            lax.dynamic_slice_in_dim(ext, c * DC + shift, WB, 1)
            for c in range(NCHUNK)
        ]
        tabs.append(jnp.stack(wins, axis=1).reshape(VOCAB * NCHUNK, WB))
    return tabs


def _sc_encode(table_a, table_b, x):
    mesh = plsc.VectorSubcoreMesh(
        core_axis_name="c", subcore_axis_name="s", num_cores=NC, num_subcores=NS
    )

    @functools.partial(
        pl.kernel,
        out_type=jax.ShapeDtypeStruct((B, DP), jnp.bfloat16),
        mesh=mesh,
        compiler_params=pltpu.CompilerParams(use_tc_tiling_on_sc=False),
        scratch_types=[
            pltpu.VMEM((BPW, L), jnp.int32),
            pltpu.VMEM((2, L), jnp.int32),
            pltpu.VMEM((2, L, WB), jnp.bfloat16),
            pltpu.VMEM((2, L, WB), jnp.bfloat16),
            pltpu.VMEM((RB, DC), jnp.bfloat16),
            pltpu.SemaphoreType.DMA,
            pltpu.SemaphoreType.DMA,
        ],
    )
    def enc_kernel(
        ta_hbm, tb_hbm, x_hbm, enc_hbm, xw, idxv, bufa, bufb, acc, sem0, sem1
    ):
        wid = lax.axis_index("s") * NC + lax.axis_index("c")
        base_b = wid * BPW
        pltpu.sync_copy(x_hbm.at[pl.ds(base_b, BPW)], xw)
        sems = (sem0, sem1)

        def fire(pb, i, c):
            idxv[pb, pl.ds(0, 16)] = xw[i, pl.ds(0, 16)] * NCHUNK + c
            idxv[pb, pl.ds(4, 16)] = xw[i, pl.ds(4, 16)] * NCHUNK + c
            pltpu.async_copy(ta_hbm.at[idxv.at[pb]], bufa.at[pb], sems[pb])
            pltpu.async_copy(tb_hbm.at[idxv.at[pb]], bufb.at[pb], sems[pb])

        def drain(pb):
            pltpu.make_async_copy(ta_hbm.at[idxv.at[pb]], bufa.at[pb], sems[pb]).wait()
            pltpu.make_async_copy(tb_hbm.at[idxv.at[pb]], bufb.at[pb], sems[pb]).wait()

        def compute(pb, r):
            NG = 16
            for g in range(DC // (32 * NG)):
                base = g * 32 * NG

                def tbody(t, accs):
                    out = []
                    for k in range(NG):
                        o = base + k * 32
                        v = bufa[pb, t, pl.ds(o, 32)]
                        v = v * bufb[pb, t + 1, pl.ds(o, 32)]
                        v = v * bufa[pb, t + 2, pl.ds(o + 2, 32)]
                        out.append(accs[k] + v)
                    return tuple(out)

                z = jnp.zeros((32,), jnp.bfloat16)
                accs = lax.fori_loop(0, _NT, tbody, (z,) * NG)
                for k in range(NG):
                    acc[r, pl.ds(base + k * 32, 32)] = accs[k]

        NR2 = RB // 2
        NU = (BPW // RB) * NCHUNK * NR2

        def unit(u):
            i8 = u // (NCHUNK * NR2)
            rem = u % (NCHUNK * NR2)
            return i8, rem // NR2, (rem % NR2) * 2

        fire(0, 0, 0)

        def body_u(u, carry):
            i8, c, r0 = unit(u)
            i = i8 * RB + r0
            fire(1, i + 1, c)
            drain(0)
            compute(0, r0)

            @pl.when(u < NU - 1)
            def _():
                ni8, nc, nr0 = unit(u + 1)
                fire(0, ni8 * RB + nr0, nc)

            drain(1)
            compute(1, r0 + 1)

            @pl.when(r0 == RB - 2)
            def _():
                row0 = pl.multiple_of(base_b + i8 * RB, RB)
                col0 = pl.multiple_of(c * DC, 256)
                pltpu.sync_copy(
                    acc, enc_hbm.at[pl.ds(row0, RB), pl.ds(col0, DC)]
                )

            return carry

        lax.fori_loop(0, NU, body_u, 0)

    return enc_kernel(table_a, table_b, x)


def _tc_search(enc, am_pad):
    BB = 128

    def body(enc_ref, am_ref, out_ref):
        am = am_ref[...]
        an = jnp.sqrt(jnp.sum(am * am, axis=1, keepdims=True)) + 1e-12
        am_n = am / an
        e = enc_ref[...].astype(jnp.float32)
        en = jnp.sqrt(jnp.sum(e * e, axis=1, keepdims=True)) + 1e-12
        s = lax.dot_general(
            e, am_n, (((1,), (1,)), ((), ())), preferred_element_type=jnp.float32
        )
        out_ref[...] = s / en

    return pl.pallas_call(
        body,
        grid=(B // BB,),
        in_specs=[
            pl.BlockSpec((BB, DP), lambda i: (i, 0)),
            pl.BlockSpec((NUM_CLASSES, DP), lambda i: (0, 0)),
        ],
        out_specs=pl.BlockSpec((BB, NUM_CLASSES), lambda i: (i, 0)),
        out_shape=jax.ShapeDtypeStruct((B, NUM_CLASSES), jnp.float32),
    )(enc, am_pad)


@jax.jit
def kernel(x, id_weight, am_weight):
    table_a, table_b = _build_tables(id_weight)
    enc = _sc_encode(table_a, table_b, x.astype(jnp.int32))
    am_pad = jnp.pad(am_weight, ((0, 0), (0, DP - D)))
    return _tc_search(enc, am_pad)


# speedup vs baseline: 2.1966x; 1.0001x over previous
import functools

import jax
import jax.numpy as jnp
import numpy as np
from jax import lax
from jax.experimental import pallas as pl
from jax.experimental.pallas import tpu as pltpu
from jax.experimental.pallas import tpu_sc as plsc

B, L, D = 1024, 20, 10000
VOCAB, NUM_CLASSES, NGRAM_N = 1000, 100, 3

NC, NS = 2, 16
NW = NC * NS
BPW = B // NW
RB = 8

NCHUNK = 5
DP = 10240
DC = DP // NCHUNK
WB = DC + 32

_NT = L - (NGRAM_N - 1)


def _build_tables(id_weight):
    wb = id_weight.astype(jnp.bfloat16)
    pad = (NCHUNK - 1) * DC + 1 + WB - 2 - D
    ext = jnp.concatenate([wb[:, D - 2 :], wb, jnp.zeros((VOCAB, pad), wb.dtype)], 1)
    tabs = []
    for shift in (0, 1):
        wins = [
            lax.dynamic_slice_in_dim(ext, c * DC + shift, WB, 1)
            for c in range(NCHUNK)
        ]
        tabs.append(jnp.stack(wins, axis=1).reshape(VOCAB * NCHUNK, WB))
    return tabs


def _sc_encode(table_a, table_b, x):
    mesh = plsc.VectorSubcoreMesh(
        core_axis_name="c", subcore_axis_name="s", num_cores=NC, num_subcores=NS
    )

    @functools.partial(
        pl.kernel,
        out_type=jax.ShapeDtypeStruct((B, DP), jnp.bfloat16),
        mesh=mesh,
        compiler_params=pltpu.CompilerParams(use_tc_tiling_on_sc=False),
        scratch_types=[
            pltpu.VMEM((BPW, L), jnp.int32),
            pltpu.VMEM((2, L), jnp.int32),
            pltpu.VMEM((2, L, WB), jnp.bfloat16),
            pltpu.VMEM((2, L, WB), jnp.bfloat16),
            pltpu.VMEM((RB, DC), jnp.bfloat16),
            pltpu.SemaphoreType.DMA,
            pltpu.SemaphoreType.DMA,
        ],
    )
    def enc_kernel(
        ta_hbm, tb_hbm, x_hbm, enc_hbm, xw, idxv, bufa, bufb, acc, sem0, sem1
    ):
        wid = lax.axis_index("s") * NC + lax.axis_index("c")
        base_b = wid * BPW
        pltpu.sync_copy(x_hbm.at[pl.ds(base_b, BPW)], xw)
        sems = (sem0, sem1)

        def fire(pb, i, c):
            idxv[pb, pl.ds(0, 16)] = xw[i, pl.ds(0, 16)] * NCHUNK + c
            idxv[pb, pl.ds(4, 16)] = xw[i, pl.ds(4, 16)] * NCHUNK + c
            pltpu.async_copy(ta_hbm.at[idxv.at[pb]], bufa.at[pb], sems[pb])
            pltpu.async_copy(tb_hbm.at[idxv.at[pb]], bufb.at[pb], sems[pb])

        def drain(pb):
            pltpu.make_async_copy(ta_hbm.at[idxv.at[pb]], bufa.at[pb], sems[pb]).wait()
            pltpu.make_async_copy(tb_hbm.at[idxv.at[pb]], bufb.at[pb], sems[pb]).wait()

        def compute(pb, r):
            NG = 8
            for g in range(DC // (32 * NG)):
                base = g * 32 * NG

                def tbody(t, accs):
                    out = []
                    for k in range(NG):
                        o = base + k * 32
                        v = bufa[pb, t, pl.ds(o, 32)]
                        v = v * bufb[pb, t + 1, pl.ds(o, 32)]
                        v = v * bufa[pb, t + 2, pl.ds(o + 2, 32)]
                        out.append(accs[k] + v)
                    return tuple(out)

                z = jnp.zeros((32,), jnp.bfloat16)
                accs = lax.fori_loop(0, _NT, tbody, (z,) * NG)
                for k in range(NG):
                    acc[r, pl.ds(base + k * 32, 32)] = accs[k]

        NR2 = RB // 2
        NU = (BPW // RB) * NCHUNK * NR2

        def unit(u):
            i8 = u // (NCHUNK * NR2)
            rem = u % (NCHUNK * NR2)
            return i8, rem // NR2, (rem % NR2) * 2

        fire(0, 0, 0)

        def body_u(u, carry):
            i8, c, r0 = unit(u)
            i = i8 * RB + r0
            fire(1, i + 1, c)
            drain(0)
            compute(0, r0)

            @pl.when(u < NU - 1)
            def _():
                ni8, nc, nr0 = unit(u + 1)
                fire(0, ni8 * RB + nr0, nc)

            drain(1)
            compute(1, r0 + 1)

            @pl.when(r0 == RB - 2)
            def _():
                row0 = pl.multiple_of(base_b + i8 * RB, RB)
                col0 = pl.multiple_of(c * DC, 256)
                pltpu.sync_copy(
                    acc, enc_hbm.at[pl.ds(row0, RB), pl.ds(col0, DC)]
                )

            return carry

        lax.fori_loop(0, NU, body_u, 0)

    return enc_kernel(table_a, table_b, x)


def _tc_search(enc, am_pad):
    BB = 128

    def body(enc_ref, am_ref, out_ref):
        am = am_ref[...]
        an = jnp.sqrt(jnp.sum(am * am, axis=1, keepdims=True)) + 1e-12
        am_n = am / an
        e = enc_ref[...].astype(jnp.float32)
        en = jnp.sqrt(jnp.sum(e * e, axis=1, keepdims=True)) + 1e-12
        s = lax.dot_general(
            e, am_n, (((1,), (1,)), ((), ())), preferred_element_type=jnp.float32
        )
        out_ref[...] = s / en

    return pl.pallas_call(
        body,
        grid=(B // BB,),
        in_specs=[
            pl.BlockSpec((BB, DP), lambda i: (i, 0)),
            pl.BlockSpec((NUM_CLASSES, DP), lambda i: (0, 0)),
        ],
        out_specs=pl.BlockSpec((BB, NUM_CLASSES), lambda i: (i, 0)),
        out_shape=jax.ShapeDtypeStruct((B, NUM_CLASSES), jnp.float32),
    )(enc, am_pad)


@jax.jit
def kernel(x, id_weight, am_weight):
    table_a, table_b = _build_tables(id_weight)
    enc = _sc_encode(table_a, table_b, x.astype(jnp.int32))
    am_pad = jnp.pad(am_weight, ((0, 0), (0, DP - D)))
    return _tc_search(enc, am_pad)
